# baseline (speedup 1.0000x reference)
import jax
import jax.numpy as jnp
from jax import lax
from jax.experimental import pallas as pl

N_NODES_ = 100000
N_TYPES_ = 64
D_ = 128
BLK_ = 2000


def _tc_body(idx_ref, w_ref, b_ref, emb_ref, oh_ref):
    idx = idx_ref[...]
    iota = lax.broadcasted_iota(jnp.int32, (BLK_, N_TYPES_), 1)
    onehot = (idx == iota).astype(jnp.float32)
    oh_ref[...] = onehot
    emb_ref[...] = jnp.dot(onehot, w_ref[...],
                           preferred_element_type=jnp.float32) + b_ref[...]


def kernel(node_atom, W, b):
    idx2 = node_atom.astype(jnp.int32).reshape(N_NODES_, 1)
    b2 = b.reshape(1, D_)
    grid = N_NODES_ // BLK_
    emb, oh = pl.pallas_call(
        _tc_body,
        grid=(grid,),
        in_specs=[
            pl.BlockSpec((BLK_, 1), lambda i: (i, 0)),
            pl.BlockSpec((N_TYPES_, D_), lambda i: (0, 0)),
            pl.BlockSpec((1, D_), lambda i: (0, 0)),
        ],
        out_specs=[
            pl.BlockSpec((BLK_, D_), lambda i: (i, 0)),
            pl.BlockSpec((BLK_, N_TYPES_), lambda i: (i, 0)),
        ],
        out_shape=[
            jax.ShapeDtypeStruct((N_NODES_, D_), jnp.float32),
            jax.ShapeDtypeStruct((N_NODES_, N_TYPES_), jnp.float32),
        ],
    )(idx2, W, b2)
    return (emb, oh, oh)


# speedup vs baseline: 1.5016x; 1.5016x over previous
import jax
import jax.numpy as jnp
from jax import lax
from jax.experimental import pallas as pl

N_NODES_ = 100000
N_TYPES_ = 64
D_ = 128
BLK_ = 2000


def _tc_body(idx_ref, t_ref, emb_ref, oh_ref):
    idx = idx_ref[0]
    iota = lax.broadcasted_iota(jnp.int32, (N_TYPES_, BLK_), 0)
    onehot_t = (idx == iota).astype(jnp.float32)
    res = lax.dot_general(
        onehot_t, t_ref[...], (((0,), (0,)), ((), ())),
        preferred_element_type=jnp.float32)
    emb_ref[...] = res[:, :D_]
    oh_ref[...] = res[:, D_:]


def kernel(node_atom, W, b):
    idx3 = node_atom.astype(jnp.int32).reshape(N_NODES_ // BLK_, 1, BLK_)
    table = jnp.concatenate(
        [W + b[None, :], jnp.eye(N_TYPES_, dtype=jnp.float32)], axis=1)
    grid = N_NODES_ // BLK_
    emb, oh = pl.pallas_call(
        _tc_body,
        grid=(grid,),
        in_specs=[
            pl.BlockSpec((1, 1, BLK_), lambda i: (i, 0, 0)),
            pl.BlockSpec((N_TYPES_, D_ + N_TYPES_), lambda i: (0, 0)),
        ],
        out_specs=[
            pl.BlockSpec((BLK_, D_), lambda i: (i, 0)),
            pl.BlockSpec((BLK_, N_TYPES_), lambda i: (i, 0)),
        ],
        out_shape=[
            jax.ShapeDtypeStruct((N_NODES_, D_), jnp.float32),
            jax.ShapeDtypeStruct((N_NODES_, N_TYPES_), jnp.float32),
        ],
    )(idx3, table)
    return (emb, oh, oh)


# speedup vs baseline: 1.7627x; 1.1739x over previous
import jax
import jax.numpy as jnp
from jax import lax
from jax.experimental import pallas as pl

N_NODES_ = 100000
N_TYPES_ = 64
D_ = 128
BLK_ = 5000


def _tc_body(idx_ref, t_ref, emb_ref, oh_ref):
    idx = idx_ref[0]
    iota = lax.broadcasted_iota(jnp.int32, (N_TYPES_, BLK_), 0)
    onehot_t = (idx == iota).astype(jnp.float32)
    res = lax.dot_general(
        onehot_t, t_ref[...], (((0,), (0,)), ((), ())),
        preferred_element_type=jnp.float32)
    emb_ref[...] = res[:, :D_]
    oh_ref[...] = res[:, D_:]


def kernel(node_atom, W, b):
    idx3 = node_atom.astype(jnp.int32).reshape(N_NODES_ // BLK_, 1, BLK_)
    table = jnp.concatenate(
        [W + b[None, :], jnp.eye(N_TYPES_, dtype=jnp.float32)], axis=1)
    grid = N_NODES_ // BLK_
    emb, oh = pl.pallas_call(
        _tc_body,
        grid=(grid,),
        in_specs=[
            pl.BlockSpec((1, 1, BLK_), lambda i: (i, 0, 0)),
            pl.BlockSpec((N_TYPES_, D_ + N_TYPES_), lambda i: (0, 0)),
        ],
        out_specs=[
            pl.BlockSpec((BLK_, D_), lambda i: (i, 0)),
            pl.BlockSpec((BLK_, N_TYPES_), lambda i: (i, 0)),
        ],
        out_shape=[
            jax.ShapeDtypeStruct((N_NODES_, D_), jnp.float32),
            jax.ShapeDtypeStruct((N_NODES_, N_TYPES_), jnp.float32),
        ],
    )(idx3, table)
    return (emb, oh, oh)
